# Initial kernel scaffold; baseline (speedup 1.0000x reference)
import jax
import jax.numpy as jnp
from jax.experimental import pallas as pl


def kernel(x, edge_index, W1, b1, W2, b2, num_ue):
    raise NotImplementedError("write your pallas kernel here")


# speedup vs baseline: 30.3593x; 30.3593x over previous
import functools

import jax
import jax.numpy as jnp
from jax import lax
from jax.experimental import pallas as pl
from jax.experimental.pallas import tpu as pltpu
from jax.experimental.pallas import tpu_sc as plsc

NC, NS, L = 2, 16, 16
NW = NC * NS

_B = 80
_IB = 25
_UE = 1024
_CAP = 768
_G2 = 64
_CR = 13
_A2R = 1040
_BR = 1024

_SC_PARAMS = pltpu.CompilerParams(use_tc_tiling_on_sc=False,
                                  needs_layout_passes=False)


def _popcount(mask):
    p = plsc.all_reduce_population_count(mask)
    return jnp.max(p) if p.ndim else p


def _rsqrt16(d):
    i = plsc.bitcast(d, jnp.int32)
    i = jnp.int32(0x5F3759DF) - lax.shift_right_logical(i, 1)
    y = plsc.bitcast(i, jnp.float32)
    for _ in range(3):
        y = y * (1.5 - 0.5 * d * y * y)
    return y


def _sc_mesh():
    return plsc.VectorSubcoreMesh(core_axis_name="c", subcore_axis_name="s")


def _make_deg(npad, e):
    rows = e // _B
    r_w = rows // NW
    outer = r_w // _IB
    seg = npad // NS

    @functools.partial(
        pl.kernel,
        out_type=jax.ShapeDtypeStruct((NC * npad,), jnp.float32),
        mesh=_sc_mesh(),
        compiler_params=_SC_PARAMS,
        scratch_types=[
            pltpu.VMEM((_IB, _B), jnp.int32),
            pltpu.VMEM((_B,), jnp.float32),
            pltpu.VMEM_SHARED((npad,), jnp.float32),
        ],
    )
    def deg_k(e3_hbm, za_hbm, out_hbm, idx_v, ones_v, deg_sh):
        c = lax.axis_index("c")
        s = lax.axis_index("s")
        w = s * NC + c
        dst2 = e3_hbm.at[1]

        def fill_ones(i, carry):
            ones_v[pl.ds(i * L, L)] = jnp.ones((L,), jnp.float32)
            return carry

        lax.fori_loop(0, _B // L, fill_ones, 0)
        pltpu.sync_copy(za_hbm, deg_sh.at[pl.ds(s * seg, seg)])
        plsc.subcore_barrier()

        row0 = w * r_w

        def outer_body(o, carry):
            pltpu.sync_copy(dst2.at[pl.ds(row0 + o * _IB, _IB)], idx_v)

            def inner(j, carry2):
                pltpu.sync_copy(ones_v, deg_sh.at[idx_v.at[j]], add=True)
                return carry2

            return lax.fori_loop(0, _IB, inner, carry)

        lax.fori_loop(0, outer, outer_body, 0)
        plsc.subcore_barrier()
        pltpu.sync_copy(deg_sh.at[pl.ds(s * seg, seg)],
                        out_hbm.at[pl.ds(c * npad + s * seg, seg)])

    return deg_k


def _make_agg1(npad, e, in_dim):
    rows = e // _B
    r_w = rows // NW
    outer = r_w // _IB
    seg = npad // NS
    nv = seg * in_dim // L

    @functools.partial(
        pl.kernel,
        out_type=(
            jax.ShapeDtypeStruct((NC * npad, in_dim), jnp.float32),
            jax.ShapeDtypeStruct((NW * (_CAP // _G2), _G2), jnp.int32),
            jax.ShapeDtypeStruct((NW * (_CAP // _G2), _G2), jnp.int32),
            jax.ShapeDtypeStruct((npad,), jnp.float32),
            jax.ShapeDtypeStruct((_UE,), jnp.float32),
            jax.ShapeDtypeStruct((NC * npad, in_dim), jnp.float32),
        ),
        mesh=_sc_mesh(),
        compiler_params=_SC_PARAMS,
        scratch_types=[
            pltpu.VMEM((_IB, _B), jnp.int32),
            pltpu.VMEM((_IB, _B), jnp.int32),
            pltpu.VMEM((_B, in_dim), jnp.float32),
            pltpu.VMEM((_CR, _G2), jnp.int32),
            pltpu.VMEM((_CR, _G2), jnp.int32),
            pltpu.VMEM((npad // NS,), jnp.float32),
            pltpu.VMEM((npad // NS,), jnp.float32),
            pltpu.VMEM((npad // NS, 8), jnp.float32),
            pltpu.VMEM((_IB, _B), jnp.int32),
            pltpu.VMEM_SHARED((npad, in_dim), jnp.float32),
        ],
    )
    def agg1_k(x_hbm, degp_hbm, e3_hbm, zb_hbm,
               t_out, srcc_out, dstc_out, norm_out, n1024_out, xs_out,
               src_i, dst_i, msg, srcc, dstc, dbuf, norm_v, xbuf,
               srcadj, agg_sh):
        c = lax.axis_index("c")
        s = lax.axis_index("s")
        w = s * NC + c
        base = s * seg
        src2 = e3_hbm.at[0]
        dst2 = e3_hbm.at[1]
        iota = lax.broadcasted_iota(jnp.int32, (L,), 0)
        lane8 = lax.shift_right_logical(iota, 3)
        col8 = lax.bitwise_and(iota, 7)

        pltpu.sync_copy(degp_hbm.at[pl.ds(base, seg)], dbuf)
        pltpu.sync_copy(degp_hbm.at[pl.ds(npad + base, seg)], norm_v)

        def mk_norm(i, carry):
            d = dbuf[pl.ds(i * L, L)] + norm_v[pl.ds(i * L, L)] + 1.0
            norm_v[pl.ds(i * L, L)] = _rsqrt16(d)
            return carry

        lax.fori_loop(0, seg // L, mk_norm, 0)

        pltpu.sync_copy(x_hbm.at[pl.ds(base, seg)], xbuf)

        def mk_xs(i, carry):
            rvec = lane8 + 2 * i
            ne = plsc.load_gather(norm_v, [rvec])
            v = plsc.load_gather(xbuf, [rvec, col8])
            plsc.store_scatter(xbuf, [rvec, col8], v * ne)
            return carry

        lax.fori_loop(0, nv, mk_xs, 0)
        pltpu.sync_copy(xbuf, xs_out.at[pl.ds(c * npad + base, seg)])

        @pl.when(c == 0)
        def _():
            pltpu.sync_copy(zb_hbm, agg_sh.at[pl.ds(base, seg)])

        @pl.when(c == 1)
        def _():
            pltpu.sync_copy(xbuf, agg_sh.at[pl.ds(base, seg)])

        @pl.when(c == 0)
        def _():
            pltpu.sync_copy(norm_v, norm_out.at[pl.ds(base, seg)])

        @pl.when((c == 0) & (s == 0))
        def _():
            pltpu.sync_copy(norm_v.at[pl.ds(0, _UE)], n1024_out)

        def prefill(i, carry):
            r = lax.shift_right_logical(i, 2)
            q = lax.bitwise_and(i, 3)
            srcc[r, pl.ds(q * L, L)] = iota + i * L
            dstc[r, pl.ds(q * L, L)] = jnp.full((L,), _UE, jnp.int32)
            return carry

        lax.fori_loop(0, _CR * (_G2 // L), prefill, 0)
        plsc.subcore_barrier()

        row0 = w * r_w

        def outer_body(o, cnt):
            pltpu.sync_copy(src2.at[pl.ds(row0 + o * _IB, _IB)], src_i)
            pltpu.sync_copy(dst2.at[pl.ds(row0 + o * _IB, _IB)], dst_i)

            def inner(j, cnt2):
                for t in range(_B // L):
                    srcadj[j, pl.ds(t * L, L)] = (
                        src_i[j, pl.ds(t * L, L)] + c * npad)
                pltpu.sync_copy(xs_out.at[srcadj.at[j]], msg)
                pltpu.sync_copy(msg, agg_sh.at[dst_i.at[j]], add=True)
                for t in range(_B // L):
                    sv = src_i[j, pl.ds(t * L, L)]
                    dv = dst_i[j, pl.ds(t * L, L)]
                    mask = dv < _UE
                    pos = plsc.cumsum(mask.astype(jnp.int32)) - 1 + cnt2
                    pr = lax.shift_right_logical(pos, 6)
                    pq = lax.bitwise_and(pos, 63)
                    plsc.store_scatter(srcc, [pr, pq], sv, mask=mask)
                    plsc.store_scatter(dstc, [pr, pq], dv, mask=mask)
                    cnt2 = cnt2 + _popcount(mask)
                return cnt2

            return lax.fori_loop(0, _IB, inner, cnt)

        lax.fori_loop(0, outer, outer_body, jnp.int32(0))
        plsc.subcore_barrier()

        pltpu.sync_copy(agg_sh.at[pl.ds(base, seg)], xbuf)

        def scale_t(i, carry):
            rvec = lane8 + 2 * i
            ne = plsc.load_gather(norm_v, [rvec])
            v = plsc.load_gather(xbuf, [rvec, col8])
            plsc.store_scatter(xbuf, [rvec, col8], v * ne)
            return carry

        lax.fori_loop(0, nv, scale_t, 0)
        pltpu.sync_copy(xbuf, t_out.at[pl.ds(c * npad + base, seg)])
        cr = _CAP // _G2
        pltpu.sync_copy(srcc.at[pl.ds(0, cr)],
                        srcc_out.at[pl.ds(w * cr, cr)])
        pltpu.sync_copy(dstc.at[pl.ds(0, cr)],
                        dstc_out.at[pl.ds(w * cr, cr)])

    return agg1_k


def _make_agg2(npad, hid):
    rows_w = _CAP // _G2
    seg2 = _A2R // NS

    @functools.partial(
        pl.kernel,
        out_type=jax.ShapeDtypeStruct((NC * _A2R, hid), jnp.float32),
        mesh=_sc_mesh(),
        compiler_params=_SC_PARAMS,
        scratch_types=[
            pltpu.VMEM((rows_w, _G2), jnp.int32),
            pltpu.VMEM((rows_w, _G2), jnp.int32),
            pltpu.VMEM((rows_w, _G2), jnp.float32),
            pltpu.VMEM((_G2, hid), jnp.float32),
            pltpu.VMEM_SHARED((_A2R, hid), jnp.float32),
        ],
    )
    def agg2_k(h_hbm, srcc_hbm, dstc_hbm, norm_hbm, zc_hbm, out_hbm,
               src_i, dst_i, nrm_i, rows_v, agg2_sh):
        c = lax.axis_index("c")
        s = lax.axis_index("s")
        w = s * NC + c

        pltpu.sync_copy(zc_hbm, agg2_sh.at[pl.ds(s * seg2, seg2)])
        pltpu.sync_copy(srcc_hbm.at[pl.ds(w * rows_w, rows_w)], src_i)
        pltpu.sync_copy(dstc_hbm.at[pl.ds(w * rows_w, rows_w)], dst_i)
        plsc.subcore_barrier()

        def body_j(j, carry):
            pltpu.sync_copy(h_hbm.at[src_i.at[j]], rows_v)
            pltpu.sync_copy(norm_hbm.at[src_i.at[j]], nrm_i.at[j])

            def scale_row(i, carry2):
                nvec = plsc.load_gather(nrm_i.at[j],
                                        [jnp.full((L,), i, jnp.int32)])
                for t in range(hid // L):
                    rows_v[i, pl.ds(t * L, L)] = rows_v[i, pl.ds(t * L, L)] * nvec
                return carry2

            lax.fori_loop(0, _G2, scale_row, 0)
            pltpu.sync_copy(rows_v, agg2_sh.at[dst_i.at[j]], add=True)
            return carry

        lax.fori_loop(0, rows_w, body_j, 0)
        plsc.subcore_barrier()
        pltpu.sync_copy(agg2_sh.at[pl.ds(s * seg2, seg2)],
                        out_hbm.at[pl.ds(c * _A2R + s * seg2, seg2)])

    return agg2_k


def _tc_hidden(t_a, t_b, w1e, b1e, npad, in_dim, hid):
    fr = npad * in_dim // 128
    ob = hid * 16
    grid = 8
    br = fr // grid

    def body(ta, tb, w1r, b1r, h_o):
        tf = ta[...] + tb[...]
        m = jnp.maximum(
            jnp.dot(tf, w1r[...], preferred_element_type=jnp.float32) + b1r[...],
            0.0)
        h_o[...] = m.reshape(br * 16, hid)

    return pl.pallas_call(
        body,
        grid=(grid,),
        in_specs=[
            pl.BlockSpec((br, 128), lambda i: (i, 0)),
            pl.BlockSpec((br, 128), lambda i: (i, 0)),
            pl.BlockSpec((128, ob), lambda i: (0, 0)),
            pl.BlockSpec((1, ob), lambda i: (0, 0)),
        ],
        out_specs=pl.BlockSpec((br * 16, hid), lambda i: (i, 0)),
        out_shape=jax.ShapeDtypeStruct((npad, hid), jnp.float32),
    )(t_a, t_b, w1e, b1e)


def _tc_out(a2a, a2b, h1, nrm1, w2, b2):
    hid = w2.shape[0]
    out = w2.shape[1]

    def body(pa, pb, hr, nr, w2r, b2r, o):
        n = nr[...]
        t = (pa[...] + pb[...]) * n + hr[...] * (n * n)
        o[...] = jnp.dot(t, w2r[...], preferred_element_type=jnp.float32) + b2r[...]

    return pl.pallas_call(
        body,
        grid=(1,),
        in_specs=[
            pl.BlockSpec((_UE, hid), lambda i: (0, 0)),
            pl.BlockSpec((_UE, hid), lambda i: (0, 0)),
            pl.BlockSpec((_UE, hid), lambda i: (0, 0)),
            pl.BlockSpec((_UE, 1), lambda i: (0, 0)),
            pl.BlockSpec((hid, out), lambda i: (0, 0)),
            pl.BlockSpec((1, out), lambda i: (0, 0)),
        ],
        out_specs=pl.BlockSpec((_UE, out), lambda i: (0, 0)),
        out_shape=jax.ShapeDtypeStruct((_UE, out), jnp.float32),
    )(a2a, a2b, h1, nrm1, w2, b2)


def kernel(x, edge_index, W1, b1, W2, b2, num_ue):
    n, in_dim = x.shape
    e = edge_index.shape[1]
    hid = W1.shape[1]
    npad = ((n + _BR - 1) // _BR) * _BR
    seg = npad // NS

    xp = jnp.pad(x.reshape(-1), (0, (npad - n) * in_dim)).reshape(npad, in_dim)
    e3 = edge_index.reshape(2, e // _B, _B)
    za = jnp.zeros((seg,), jnp.float32)
    zb = jnp.zeros((seg, in_dim), jnp.float32)
    zc = jnp.zeros((_A2R // NS, hid), jnp.float32)
    w1e = jnp.kron(jnp.eye(16, dtype=jnp.float32), W1)
    b1e = jnp.tile(b1, 16).reshape(1, hid * 16)

    degp = _make_deg(npad, e)(e3, za)
    t2, srcc, dstc, nrm, n1024, _xs = _make_agg1(npad, e, in_dim)(
        xp, degp, e3, zb)
    fr = npad * in_dim // 128
    tf = t2.reshape(NC * fr, 128)
    h = _tc_hidden(tf[:fr], tf[fr:], w1e, b1e, npad, in_dim, hid)
    a2f = _make_agg2(npad, hid)(h, srcc, dstc, nrm, zc)
    ue = _tc_out(a2f[:_UE], a2f[_A2R:_A2R + _UE], h[:_UE],
                 n1024.reshape(_UE, 1), W2, b2.reshape(1, W2.shape[1]))
    return ue
